# Initial kernel scaffold; baseline (speedup 1.0000x reference)
import jax
import jax.numpy as jnp
from jax.experimental import pallas as pl


def kernel(peak_map, logit_thresh):
    raise NotImplementedError("write your pallas kernel here")


# speedup vs baseline: 1.2050x; 1.2050x over previous
import functools

import jax
import jax.numpy as jnp
from jax import lax
from jax.experimental import pallas as pl
from jax.experimental.pallas import tpu as pltpu
from jax.experimental.pallas import tpu_sc as plsc

ROWS = 16
COLS = 20000
HALF = COLS // 2
NC = 2
NS = 16
L = 16
SHARP = 10.0
NEG = float("-inf")

XBUF = HALF + 2 * L


def _body(x_hbm, lg_hbm, sp_hbm, mk_hbm, pv_hbm, xb, spb, mkb, pvb, lgb):
    cid = lax.axis_index("c")
    sid = lax.axis_index("s")
    wid = sid * NC + cid
    row = wid // 2
    half = wid % 2

    pltpu.sync_copy(lg_hbm, lgb)
    logit = lgb[...]
    thresh = 1.0 / (1.0 + jnp.exp(-logit))

    src = row * COLS + (HALF - L) * half
    dst = L * (1 - half)
    pltpu.sync_copy(x_hbm.at[pl.ds(src, HALF + L)], xb.at[pl.ds(dst, HALF + L)])
    xb[pl.ds((HALF + L) * half, L)] = jnp.full((L,), NEG, jnp.float32)

    def step(i, carry):
        base = L + i * L
        x = xb[pl.ds(base, L)]
        a = jnp.maximum(xb[pl.ds(base - 2, L)], xb[pl.ds(base - 1, L)])
        b = jnp.maximum(xb[pl.ds(base + 1, L)], xb[pl.ds(base + 2, L)])
        pooled = jnp.maximum(x, jnp.maximum(a, b))
        gate = 1.0 / (1.0 + jnp.exp(SHARP * (thresh - x)))
        lm = 1.0 / (1.0 + jnp.exp(SHARP * (pooled - x)))
        sp = x * gate * lm
        keep = sp >= thresh
        o = i * L
        spb[pl.ds(o, L)] = sp
        mkb[pl.ds(o, L)] = jnp.where(keep, 1.0, 0.0).astype(jnp.float32)
        pvb[pl.ds(o, L)] = jnp.where(keep, x, 0.0)
        return carry

    lax.fori_loop(0, HALF // L, step, 0)

    out = pl.ds(row * COLS + half * HALF, HALF)
    pltpu.sync_copy(spb, sp_hbm.at[out])
    pltpu.sync_copy(mkb, mk_hbm.at[out])
    pltpu.sync_copy(pvb, pv_hbm.at[out])


@jax.jit
def _run(peak_map, logit_vec):
    mesh = plsc.VectorSubcoreMesh(
        core_axis_name="c", subcore_axis_name="s", num_cores=NC, num_subcores=NS
    )
    f = pl.kernel(
        _body,
        out_type=(
            jax.ShapeDtypeStruct((ROWS * COLS,), jnp.float32),
            jax.ShapeDtypeStruct((ROWS * COLS,), jnp.float32),
            jax.ShapeDtypeStruct((ROWS * COLS,), jnp.float32),
        ),
        mesh=mesh,
        scratch_types=[
            pltpu.VMEM((XBUF,), jnp.float32),
            pltpu.VMEM((HALF,), jnp.float32),
            pltpu.VMEM((HALF,), jnp.float32),
            pltpu.VMEM((HALF,), jnp.float32),
            pltpu.VMEM((L,), jnp.float32),
        ],
    )
    return f(peak_map.reshape(ROWS * COLS), logit_vec)


def kernel(peak_map, logit_thresh):
    logit_vec = jnp.broadcast_to(logit_thresh.astype(jnp.float32), (L,))
    sp, mk, pv = _run(peak_map, logit_vec)
    shape = (ROWS, COLS)
    return sp.reshape(shape), (mk != 0.0).reshape(shape), pv.reshape(shape)


# speedup vs baseline: 1.2343x; 1.0244x over previous
import functools

import jax
import jax.numpy as jnp
from jax import lax
from jax.experimental import pallas as pl
from jax.experimental.pallas import tpu as pltpu
from jax.experimental.pallas import tpu_sc as plsc

ROWS = 16
COLS = 20000
HALF = COLS // 2
NC = 2
NS = 16
L = 16
SHARP = 10.0
NEG = float("-inf")

XBUF = HALF + 2 * L


def _body(x_hbm, lg_hbm, sp_hbm, mk_hbm, pv_hbm, xb, spb, mkb, pvb, lgb):
    cid = lax.axis_index("c")
    sid = lax.axis_index("s")
    wid = sid * NC + cid
    row = wid // 2
    half = wid % 2

    pltpu.sync_copy(lg_hbm, lgb)
    logit = lgb[...]
    thresh = 1.0 / (1.0 + jnp.exp(-logit))

    src = row * COLS + (HALF - L) * half
    dst = L * (1 - half)
    pltpu.sync_copy(x_hbm.at[pl.ds(src, HALF + L)], xb.at[pl.ds(dst, HALF + L)])
    xb[pl.ds((HALF + L) * half, L)] = jnp.full((L,), NEG, jnp.float32)

    @plsc.parallel_loop(0, HALF, step=L, unroll=8)
    def step(o):
        base = L + o
        x = xb[pl.ds(base, L)]
        a = jnp.maximum(xb[pl.ds(base - 2, L)], xb[pl.ds(base - 1, L)])
        b = jnp.maximum(xb[pl.ds(base + 1, L)], xb[pl.ds(base + 2, L)])
        pooled = jnp.maximum(x, jnp.maximum(a, b))
        ea = jnp.exp(SHARP * (thresh - x))
        eb = jnp.exp(SHARP * (pooled - x))
        sp = x / (1.0 + ea + eb + ea * eb)
        keep = sp >= thresh
        spb[pl.ds(o, L)] = sp
        mkb[pl.ds(o, L)] = jnp.where(keep, 1.0, 0.0).astype(jnp.float32)
        pvb[pl.ds(o, L)] = jnp.where(keep, x, 0.0)

    out = pl.ds(row * COLS + half * HALF, HALF)
    pltpu.sync_copy(spb, sp_hbm.at[out])
    pltpu.sync_copy(mkb, mk_hbm.at[out])
    pltpu.sync_copy(pvb, pv_hbm.at[out])


@jax.jit
def _run(peak_map, logit_vec):
    mesh = plsc.VectorSubcoreMesh(
        core_axis_name="c", subcore_axis_name="s", num_cores=NC, num_subcores=NS
    )
    f = pl.kernel(
        _body,
        out_type=(
            jax.ShapeDtypeStruct((ROWS * COLS,), jnp.float32),
            jax.ShapeDtypeStruct((ROWS * COLS,), jnp.float32),
            jax.ShapeDtypeStruct((ROWS * COLS,), jnp.float32),
        ),
        mesh=mesh,
        scratch_types=[
            pltpu.VMEM((XBUF,), jnp.float32),
            pltpu.VMEM((HALF,), jnp.float32),
            pltpu.VMEM((HALF,), jnp.float32),
            pltpu.VMEM((HALF,), jnp.float32),
            pltpu.VMEM((L,), jnp.float32),
        ],
    )
    return f(peak_map.reshape(ROWS * COLS), logit_vec)


def kernel(peak_map, logit_thresh):
    logit_vec = jnp.broadcast_to(logit_thresh.astype(jnp.float32), (L,))
    sp, mk, pv = _run(peak_map, logit_vec)
    shape = (ROWS, COLS)
    return sp.reshape(shape), (mk != 0.0).reshape(shape), pv.reshape(shape)
